# baseline (speedup 1.0000x reference)
import math
import functools

import jax
import jax.numpy as jnp
from jax import lax
from jax.experimental import pallas as pl
from jax.experimental.pallas import tpu as pltpu
from jax.experimental.pallas import tpu_sc as plsc

D_MODEL = 128
BATCH = 16384
N_BASE = 10000.0
SCALE = 0.2

TC_ROWS = 8192
SC_ROWS = BATCH - TC_ROWS
_TC_BLK = 512

_INFO = plsc.get_sparse_core_info()
_NC = _INFO.num_cores
_NS = _INFO.num_subcores
_LANES = _INFO.num_lanes
_NW = _NC * _NS
_B_PER_W = SC_ROWS // _NW
_CHUNK = 64
_N_CHUNK = _B_PER_W // _CHUNK
_VPR = D_MODEL // _LANES


def _make_sc_gather():
    mesh = plsc.VectorSubcoreMesh(core_axis_name="c", subcore_axis_name="s")

    @functools.partial(
        pl.kernel,
        mesh=mesh,
        out_type=jax.ShapeDtypeStruct((SC_ROWS, D_MODEL), jnp.float32),
        scratch_types=[
            pltpu.VMEM((_B_PER_W,), jnp.int32),
            pltpu.VMEM((_N_CHUNK, _CHUNK, D_MODEL), jnp.float32),
        ]
        + [pltpu.SemaphoreType.DMA] * (_N_CHUNK + 1),
    )
    def sc_gather(table_hbm, idx_hbm, out_hbm, idx_v, rows_v, *sems):
        gsems, ssem = sems[:_N_CHUNK], sems[_N_CHUNK]
        wid = lax.axis_index("s") * _NC + lax.axis_index("c")
        base = wid * _B_PER_W
        pltpu.sync_copy(idx_hbm.at[pl.ds(wid * _B_PER_W, _B_PER_W)], idx_v)
        gathers = [
            pltpu.async_copy(
                table_hbm.at[idx_v.at[pl.ds(j * _CHUNK, _CHUNK)]],
                rows_v.at[j],
                gsems[j],
            )
            for j in range(_N_CHUNK)
        ]
        stores = []
        for j in range(_N_CHUNK):
            gathers[j].wait()

            def scale_rows(r, _, j=j):
                for rr in range(2):
                    for c in range(_VPR):
                        sl = pl.ds(c * _LANES, _LANES)
                        rows_v[j, r * 2 + rr, sl] = rows_v[j, r * 2 + rr, sl] * SCALE
                return _

            lax.fori_loop(0, _CHUNK // 2, scale_rows, None)
            stores.append(
                pltpu.async_copy(
                    rows_v.at[j], out_hbm.at[pl.ds(base + j * _CHUNK, _CHUNK)], ssem
                )
            )
        for s in stores:
            s.wait()

    return sc_gather


_SC_GATHER = _make_sc_gather()


def _tc_body(t_ref, dfull_ref, out_ref):
    tv = t_ref[...]
    arg = tv * dfull_ref[...]
    par = lax.broadcasted_iota(jnp.int32, (_TC_BLK, D_MODEL), 1) % 2
    out_ref[...] = jnp.where(par == 0, jnp.sin(arg), jnp.cos(arg)) * SCALE


def _tc_call(tf, dfull):
    return pl.pallas_call(
        _tc_body,
        grid=(TC_ROWS // _TC_BLK,),
        in_specs=[
            pl.BlockSpec((_TC_BLK, 1), lambda i: (i, 0)),
            pl.BlockSpec((1, D_MODEL), lambda i: (0, 0)),
        ],
        out_specs=pl.BlockSpec((_TC_BLK, D_MODEL), lambda i: (i, 0)),
        out_shape=jax.ShapeDtypeStruct((TC_ROWS, D_MODEL), jnp.float32),
    )(tf, dfull)


def kernel(pe, t):
    div = jnp.exp(
        jnp.arange(0, D_MODEL, 2, dtype=jnp.float32) * (-math.log(N_BASE) / D_MODEL)
    )
    dfull = jnp.repeat(div, 2).reshape(1, D_MODEL)
    tf = t[:TC_ROWS].astype(jnp.float32).reshape(TC_ROWS, 1)
    tc_out = _tc_call(tf, dfull)
    sc_out = _SC_GATHER(pe, t[TC_ROWS:])
    return jnp.concatenate([tc_out, sc_out], axis=0)


# speedup vs baseline: 1.6800x; 1.6800x over previous
import functools

import jax
import jax.numpy as jnp
from jax import lax
from jax.experimental import pallas as pl
from jax.experimental.pallas import tpu as pltpu
from jax.experimental.pallas import tpu_sc as plsc

D_MODEL = 128
BATCH = 16384
SCALE = 0.2

_INFO = plsc.get_sparse_core_info()
_NC = _INFO.num_cores
_NS = _INFO.num_subcores
_LANES = _INFO.num_lanes
_NW = _NC * _NS
_B_PER_W = BATCH // _NW
_CHUNK = 64
_N_CHUNK = _B_PER_W // _CHUNK
_VPR = D_MODEL // _LANES


def _make_sc_gather():
    mesh = plsc.VectorSubcoreMesh(core_axis_name="c", subcore_axis_name="s")

    @functools.partial(
        pl.kernel,
        mesh=mesh,
        out_type=jax.ShapeDtypeStruct((BATCH, D_MODEL), jnp.float32),
        scratch_types=[
            pltpu.VMEM((_B_PER_W,), jnp.int32),
            pltpu.VMEM((_N_CHUNK, _CHUNK, D_MODEL), jnp.float32),
        ]
        + [pltpu.SemaphoreType.DMA] * (_N_CHUNK + 1),
    )
    def sc_gather(table_hbm, idx_hbm, out_hbm, idx_v, rows_v, *sems):
        gsems, ssem = sems[:_N_CHUNK], sems[_N_CHUNK]
        wid = lax.axis_index("s") * _NC + lax.axis_index("c")
        base = wid * _B_PER_W
        pltpu.sync_copy(idx_hbm.at[pl.ds(wid * _B_PER_W, _B_PER_W)], idx_v)
        gathers = [
            pltpu.async_copy(
                table_hbm.at[idx_v.at[pl.ds(j * _CHUNK, _CHUNK)]],
                rows_v.at[j],
                gsems[j],
            )
            for j in range(_N_CHUNK)
        ]
        stores = []
        for j in range(_N_CHUNK):
            gathers[j].wait()

            def scale_rows(r, _, j=j):
                for rr in range(4):
                    for c in range(_VPR):
                        sl = pl.ds(c * _LANES, _LANES)
                        rows_v[j, r * 4 + rr, sl] = rows_v[j, r * 4 + rr, sl] * SCALE
                return _

            lax.fori_loop(0, _CHUNK // 4, scale_rows, None)
            stores.append(
                pltpu.async_copy(
                    rows_v.at[j], out_hbm.at[pl.ds(base + j * _CHUNK, _CHUNK)], ssem
                )
            )
        for s in stores:
            s.wait()

    return sc_gather


_SC_GATHER = _make_sc_gather()


def kernel(pe, t):
    return _SC_GATHER(pe, t)


# speedup vs baseline: 1.7113x; 1.0186x over previous
import functools

import jax
import jax.numpy as jnp
from jax import lax
from jax.experimental import pallas as pl
from jax.experimental.pallas import tpu as pltpu
from jax.experimental.pallas import tpu_sc as plsc

D_MODEL = 128
BATCH = 16384
SCALE = 0.2

_INFO = plsc.get_sparse_core_info()
_NC = _INFO.num_cores
_NS = _INFO.num_subcores
_LANES = _INFO.num_lanes
_NW = _NC * _NS
_B_PER_W = BATCH // _NW
_CHUNK = 64
_N_CHUNK = _B_PER_W // _CHUNK
_VPR = D_MODEL // _LANES


def _make_sc_gather():
    mesh = plsc.VectorSubcoreMesh(core_axis_name="c", subcore_axis_name="s")

    @functools.partial(
        pl.kernel,
        mesh=mesh,
        out_type=jax.ShapeDtypeStruct((BATCH, D_MODEL), jnp.float32),
        scratch_types=[
            pltpu.VMEM((_B_PER_W,), jnp.int32),
            pltpu.VMEM((_N_CHUNK, _CHUNK, D_MODEL), jnp.float32),
        ]
        + [pltpu.SemaphoreType.DMA] * (_N_CHUNK + 1),
    )
    def sc_gather(table_hbm, idx_hbm, out_hbm, idx_v, rows_v, *sems):
        gsems, ssem = sems[:_N_CHUNK], sems[_N_CHUNK]
        wid = lax.axis_index("s") * _NC + lax.axis_index("c")
        base = wid * _B_PER_W
        pltpu.sync_copy(idx_hbm.at[pl.ds(wid * _B_PER_W, _B_PER_W)], idx_v)
        gathers = [
            pltpu.async_copy(
                table_hbm.at[idx_v.at[pl.ds(j * _CHUNK, _CHUNK)]],
                rows_v.at[j],
                gsems[j],
            )
            for j in range(_N_CHUNK)
        ]
        stores = []
        for j in range(_N_CHUNK):
            gathers[j].wait()

            def scale_rows(r, _, j=j):
                for rr in range(2):
                    for c in range(_VPR):
                        sl = pl.ds(c * _LANES, _LANES)
                        rows_v[j, r * 2 + rr, sl] = rows_v[j, r * 2 + rr, sl] * SCALE
                return _

            lax.fori_loop(0, _CHUNK // 2, scale_rows, None)
            stores.append(
                pltpu.async_copy(
                    rows_v.at[j], out_hbm.at[pl.ds(base + j * _CHUNK, _CHUNK)], ssem
                )
            )
        for s in stores:
            s.wait()

    return sc_gather


_SC_GATHER = _make_sc_gather()


def kernel(pe, t):
    return _SC_GATHER(pe, t)
